# Initial kernel scaffold; baseline (speedup 1.0000x reference)
import jax
import jax.numpy as jnp
from jax.experimental import pallas as pl


def kernel(x, table, W):
    raise NotImplementedError("write your pallas kernel here")


# speedup vs baseline: 7.9851x; 7.9851x over previous
import functools

import jax
import jax.numpy as jnp
from jax import lax
from jax.experimental import pallas as pl
from jax.experimental.pallas import tpu as pltpu
from jax.experimental.pallas import tpu_sc as plsc

VOCAB = 1000000
EMBED_DIM = 16
BATCH = 16384
HIST_LEN = 200

NC = 2
NS = 16
NW = NC * NS
ROWS_PER_W = BATCH // NW
GROUPS = ROWS_PER_W // 128
CHUNK = 50
CHUNKS_PER_GROUP = HIST_LEN // CHUNK


def _vtab_body(t_ref, w_ref, o_ref):
    o_ref[...] = jnp.dot(t_ref[...], w_ref[...],
                         preferred_element_type=jnp.float32)


def _compute_vtab(table, W):
    tbl2 = table.reshape(VOCAB // 8, 128)
    wbd = (jnp.eye(8, dtype=jnp.float32)[:, None, :]
           * W.reshape(EMBED_DIM)[None, :, None]).reshape(128, 8)
    rows = VOCAB // 8
    block = 5000
    v2 = pl.pallas_call(
        _vtab_body,
        grid=(rows // block,),
        in_specs=[
            pl.BlockSpec((block, 128), lambda i: (i, 0)),
            pl.BlockSpec((128, 8), lambda i: (0, 0)),
        ],
        out_specs=pl.BlockSpec((block, 8), lambda i: (i, 0)),
        out_shape=jax.ShapeDtypeStruct((rows, 8), jnp.float32),
    )(tbl2, wbd)
    return v2.reshape(VOCAB)


def _sc_body(xt_hbm, v_hbm, out_hbm, idx_v, vals_v, acc_v, sem):
    c = lax.axis_index("c")
    s = lax.axis_index("s")
    wid = s * NC + c
    for g in range(GROUPS):
        def chunk(i, accs):
            pltpu.sync_copy(
                xt_hbm.at[wid,
                          pl.ds((g * HIST_LEN + i * CHUNK) * 128,
                                CHUNK * 128)],
                idx_v)
            pltpu.async_copy(v_hbm.at[idx_v], vals_v, sem).wait()
            new = []
            for t in range(8):
                a = accs[t]
                for j in range(CHUNK):
                    a = a + vals_v[pl.ds(j * 128 + t * 16, 16)]
                new.append(a)
            return tuple(new)
        zero = jnp.zeros((16,), jnp.float32)
        accs = lax.fori_loop(0, CHUNKS_PER_GROUP, chunk, (zero,) * 8)
        scale = jnp.float32(1.0 / HIST_LEN)
        for t in range(8):
            acc_v[pl.ds(t * 16, 16)] = accs[t] * scale
        pltpu.sync_copy(acc_v,
                        out_hbm.at[pl.ds(wid * ROWS_PER_W + g * 128, 128)])


_sc_gather_sum = functools.partial(
    pl.kernel,
    out_type=jax.ShapeDtypeStruct((BATCH,), jnp.float32),
    mesh=plsc.VectorSubcoreMesh(core_axis_name="c", subcore_axis_name="s"),
    scratch_types=[
        pltpu.VMEM((CHUNK * 128,), jnp.int32),
        pltpu.VMEM((CHUNK * 128,), jnp.float32),
        pltpu.VMEM((128,), jnp.float32),
        pltpu.SemaphoreType.DMA,
    ],
)(_sc_body)


@jax.jit
def kernel(x, table, W):
    v = _compute_vtab(table, W)
    xt = (x.astype(jnp.int32)
           .reshape(NW, GROUPS, 128, HIST_LEN)
           .transpose(0, 1, 3, 2)
           .reshape(NW, GROUPS * HIST_LEN * 128))
    out = _sc_gather_sum(xt, v)
    return out.reshape(BATCH, 1)
